# Initial kernel scaffold; baseline (speedup 1.0000x reference)
import jax
import jax.numpy as jnp
from jax.experimental import pallas as pl


def kernel(x, router_w, router_b, W1, b1, W2, b2, lin_w, lin_b):
    raise NotImplementedError("write your pallas kernel here")


# speedup vs baseline: 2.7042x; 2.7042x over previous
import functools

import jax
import jax.numpy as jnp
from jax import lax
from jax.experimental import pallas as pl
from jax.experimental.pallas import tpu as pltpu

_B, _S, _D, _H, _E, _C = 2, 2048, 1024, 2048, 4, 1000
_N = _B * _S
_EP = 128
_RB = 512
_NRB = _N // _RB
_TB = 256
_NBLK = _N // _TB
_NWORK = _NBLK + _E - 1
_SB = 256
_NSB = _S // _SB


def _router_body(x_ref, w_ref, b_ref, eid_ref, gate_ref, cnt_ref, aux_ref,
                 acc_ref):
    i = pl.program_id(0)

    @pl.when(i == 0)
    def _():
        acc_ref[...] = jnp.zeros_like(acc_ref)

    xb = x_ref[...]
    logits = jnp.dot(xb, w_ref[...], preferred_element_type=jnp.float32)
    logits = logits + b_ref[...]
    m = jnp.max(logits, axis=-1, keepdims=True)
    p = jnp.exp(logits - m)
    s = jnp.sum(p, axis=-1, keepdims=True)
    probs = p / s
    eid = jnp.argmax(logits, axis=-1).astype(jnp.int32)
    gate = 1.0 / s[:, 0]
    eid_ref[0, 0, :] = eid
    gate_ref[0, 0, :] = gate
    onehot = (lax.broadcasted_iota(jnp.int32, (_RB, _EP), 1)
              == eid[:, None]).astype(jnp.float32)
    acc_ref[0:1, :] += jnp.sum(probs, axis=0, keepdims=True)
    acc_ref[1:2, :] += jnp.sum(onehot, axis=0, keepdims=True)

    @pl.when(i == _NRB - 1)
    def _():
        cnt_ref[...] = acc_ref[1:2, :]
        aux = (_E / (_N * _N)) * jnp.sum(acc_ref[0:1, :] * acc_ref[1:2, :])
        aux_ref[...] = aux * jnp.ones((1, _EP), jnp.float32)


def _run_router(x_flat, router_w, router_b):
    wp = jnp.pad(router_w, ((0, 0), (0, _EP - _E)))
    bp = jnp.full((1, _EP), -jnp.inf, jnp.float32).at[0, :_E].set(router_b)
    eid3, gate3, cnt, aux = pl.pallas_call(
        _router_body,
        grid=(_NRB,),
        in_specs=[
            pl.BlockSpec((_RB, _D), lambda i: (i, 0)),
            pl.BlockSpec((_D, _EP), lambda i: (0, 0)),
            pl.BlockSpec((1, _EP), lambda i: (0, 0)),
        ],
        out_specs=[
            pl.BlockSpec((1, 1, _RB), lambda i: (i, 0, 0)),
            pl.BlockSpec((1, 1, _RB), lambda i: (i, 0, 0)),
            pl.BlockSpec((1, _EP), lambda i: (0, 0)),
            pl.BlockSpec((1, _EP), lambda i: (0, 0)),
        ],
        out_shape=[
            jax.ShapeDtypeStruct((_NRB, 1, _RB), jnp.int32),
            jax.ShapeDtypeStruct((_NRB, 1, _RB), jnp.float32),
            jax.ShapeDtypeStruct((1, _EP), jnp.float32),
            jax.ShapeDtypeStruct((1, _EP), jnp.float32),
        ],
        scratch_shapes=[pltpu.VMEM((8, _EP), jnp.float32)],
    )(x_flat, wp, bp)
    return eid3.reshape(_N), gate3.reshape(_N), cnt, aux[0, 0]


def _build_work_table(counts):
    off = jnp.concatenate(
        [jnp.zeros((1,), jnp.int32), jnp.cumsum(counts, dtype=jnp.int32)])
    bidx = jnp.arange(_NBLK, dtype=jnp.int32)[:, None]
    eidx = jnp.arange(_E, dtype=jnp.int32)[None, :]
    blk_lo = bidx * _TB
    blk_hi = blk_lo + _TB
    g_lo = off[:-1][None, :]
    g_hi = off[1:][None, :]
    active = (g_lo < blk_hi) & (g_hi > blk_lo)
    firsts = active & (jnp.cumsum(active.astype(jnp.int32), axis=1) == 1)
    afl = active.reshape(-1)
    rank = jnp.cumsum(afl.astype(jnp.int32)) - 1
    slot = jnp.where(afl, rank, _NWORK)
    bb = jnp.broadcast_to(bidx, (_NBLK, _E)).reshape(-1)
    ee = jnp.broadcast_to(eidx, (_NBLK, _E)).reshape(-1)
    last_e = (jnp.searchsorted(off, _N - 1, side='right') - 1).astype(jnp.int32)
    wb = jnp.full((_NWORK,), _NBLK - 1, jnp.int32).at[slot].set(bb, mode='drop')
    wef = jnp.full((_NWORK,), last_e, jnp.int32).at[slot].set(ee, mode='drop')
    wem = jnp.full((_NWORK,), -1, jnp.int32).at[slot].set(ee, mode='drop')
    wfirst = jnp.zeros((_NWORK,), jnp.int32).at[slot].set(
        firsts.reshape(-1).astype(jnp.int32), mode='drop')
    return jnp.stack([wb, wef, wem, wfirst])


def _ffn_body(tbl_ref, xs_ref, w1_ref, b1_ref, w2_ref, b2_ref, eid_ref,
              gate_ref, out_ref):
    w = pl.program_id(0)
    emask = tbl_ref[2, w]
    first = tbl_ref[3, w]

    @pl.when(emask >= 0)
    def _():
        xb = xs_ref[...]
        h = jnp.dot(xb, w1_ref[0], preferred_element_type=jnp.float32)
        h = jnp.maximum(h + b1_ref[0, 0], 0.0)
        y = jnp.dot(h, w2_ref[0], preferred_element_type=jnp.float32)
        y = y + b2_ref[0, 0]
        wt = jnp.where(eid_ref[0, 0, :] == emask, gate_ref[0, 0, :], 0.0)
        contrib = y * wt[:, None]

        @pl.when(first == 1)
        def _():
            out_ref[...] = contrib

        @pl.when(first == 0)
        def _():
            out_ref[...] += contrib


def _run_ffn(xs, W1, b1, W2, b2, eid_s, gate_s, tbl):
    grid_spec = pltpu.PrefetchScalarGridSpec(
        num_scalar_prefetch=1,
        grid=(_NWORK,),
        in_specs=[
            pl.BlockSpec((_TB, _D), lambda w, tbl: (tbl[0, w], 0)),
            pl.BlockSpec((1, _D, _H), lambda w, tbl: (tbl[1, w], 0, 0)),
            pl.BlockSpec((1, 1, _H), lambda w, tbl: (tbl[1, w], 0, 0)),
            pl.BlockSpec((1, _H, _D), lambda w, tbl: (tbl[1, w], 0, 0)),
            pl.BlockSpec((1, 1, _D), lambda w, tbl: (tbl[1, w], 0, 0)),
            pl.BlockSpec((1, 1, _TB), lambda w, tbl: (tbl[0, w], 0, 0)),
            pl.BlockSpec((1, 1, _TB), lambda w, tbl: (tbl[0, w], 0, 0)),
        ],
        out_specs=pl.BlockSpec((_TB, _D), lambda w, tbl: (tbl[0, w], 0)),
    )
    return pl.pallas_call(
        _ffn_body,
        grid_spec=grid_spec,
        out_shape=jax.ShapeDtypeStruct((_N, _D), jnp.float32),
    )(tbl, xs, W1, b1.reshape(_E, 1, _H), W2, b2.reshape(_E, 1, _D),
      eid_s.reshape(_NBLK, 1, _TB), gate_s.reshape(_NBLK, 1, _TB))


def _cls_body(mo_ref, lw_ref, lb_ref, out_ref, acc_ref):
    i = pl.program_id(0)

    @pl.when(i == 0)
    def _():
        acc_ref[...] = jnp.zeros_like(acc_ref)

    acc_ref[...] += jnp.sum(mo_ref[...], axis=1)

    @pl.when(i == _NSB - 1)
    def _():
        pooled = acc_ref[...] * (1.0 / _S)
        out_ref[...] = (jnp.dot(pooled, lw_ref[...],
                                preferred_element_type=jnp.float32)
                        + lb_ref[...])


def _run_classifier(moe_out, lin_w, lin_b):
    return pl.pallas_call(
        _cls_body,
        grid=(_NSB,),
        in_specs=[
            pl.BlockSpec((_B, _SB, _D), lambda i: (0, i, 0)),
            pl.BlockSpec((_D, _C), lambda i: (0, 0)),
            pl.BlockSpec((1, _C), lambda i: (0, 0)),
        ],
        out_specs=pl.BlockSpec((_B, _C), lambda i: (0, 0)),
        out_shape=jax.ShapeDtypeStruct((_B, _C), jnp.float32),
        scratch_shapes=[pltpu.VMEM((_B, _D), jnp.float32)],
    )(moe_out, lin_w, lin_b.reshape(1, _C))


def kernel(x, router_w, router_b, W1, b1, W2, b2, lin_w, lin_b):
    x_flat = x.reshape(_N, _D)
    eid, gate, cnt, aux = _run_router(x_flat, router_w, router_b)

    counts = cnt[0, :_E].astype(jnp.int32)
    perm = jnp.argsort(eid).astype(jnp.int32)
    inv_perm = jnp.zeros((_N,), jnp.int32).at[perm].set(
        jnp.arange(_N, dtype=jnp.int32))
    tbl = _build_work_table(counts)

    xs = jnp.take(x_flat, perm, axis=0)
    eid_s = jnp.take(eid, perm)
    gate_s = jnp.take(gate, perm)

    ys = _run_ffn(xs, W1, b1, W2, b2, eid_s, gate_s, tbl)

    moe_flat = jnp.take(ys, inv_perm, axis=0)
    moe_out = moe_flat.reshape(_B, _S, _D)

    logits = _run_classifier(moe_out, lin_w, lin_b)
    return (logits, moe_out, aux)


# speedup vs baseline: 2.8461x; 1.0525x over previous
import functools

import jax
import jax.numpy as jnp
from jax import lax
from jax.experimental import pallas as pl
from jax.experimental.pallas import tpu as pltpu
from jax.experimental.pallas import tpu_sc as plsc

_B, _S, _D, _H, _E, _C = 2, 2048, 1024, 2048, 4, 1000
_N = _B * _S
_EP = 128
_RB = 512
_NRB = _N // _RB
_TB = 256
_NBLK = _N // _TB
_NWORK = _NBLK + _E - 1
_SB = 256
_NSB = _S // _SB

_NC = 2
_NS = 16
_NW = _NC * _NS
_RPW = _N // _NW
_GCH = 32
_NCHK = _RPW // _GCH


def _router_body(x_ref, w_ref, b_ref, eid_ref, gate_ref, rank_ref, cnt_ref,
                 aux_ref, acc_ref):
    i = pl.program_id(0)

    @pl.when(i == 0)
    def _():
        acc_ref[...] = jnp.zeros_like(acc_ref)

    xb = x_ref[...]
    logits = jnp.dot(xb, w_ref[...], preferred_element_type=jnp.float32)
    logits = logits + b_ref[...]
    m = jnp.max(logits, axis=-1, keepdims=True)
    p = jnp.exp(logits - m)
    s = jnp.sum(p, axis=-1, keepdims=True)
    probs = p / s
    eid = jnp.argmax(logits, axis=-1).astype(jnp.int32)
    gate = 1.0 / s[:, 0]
    eid_ref[0, 0, :] = eid
    gate_ref[0, 0, :] = gate
    onehot = (lax.broadcasted_iota(jnp.int32, (_RB, _EP), 1)
              == eid[:, None]).astype(jnp.float32)
    tri = (lax.broadcasted_iota(jnp.int32, (_RB, _RB), 0)
           > lax.broadcasted_iota(jnp.int32, (_RB, _RB), 1)).astype(jnp.float32)
    prefix = jnp.dot(tri, onehot, preferred_element_type=jnp.float32)
    rank = jnp.sum(onehot * (acc_ref[1:2, :] + prefix), axis=1)
    rank_ref[0, 0, :] = rank.astype(jnp.int32)
    acc_ref[0:1, :] += jnp.sum(probs, axis=0, keepdims=True)
    acc_ref[1:2, :] += jnp.sum(onehot, axis=0, keepdims=True)

    @pl.when(i == _NRB - 1)
    def _():
        cnt_ref[...] = acc_ref[1:2, :]
        aux = (_E / (_N * _N)) * jnp.sum(acc_ref[0:1, :] * acc_ref[1:2, :])
        aux_ref[...] = aux * jnp.ones((1, _EP), jnp.float32)


def _run_router(x_flat, router_w, router_b):
    wp = jnp.pad(router_w, ((0, 0), (0, _EP - _E)))
    bp = jnp.full((1, _EP), -jnp.inf, jnp.float32).at[0, :_E].set(router_b)
    eid3, gate3, rank3, cnt, aux = pl.pallas_call(
        _router_body,
        grid=(_NRB,),
        in_specs=[
            pl.BlockSpec((_RB, _D), lambda i: (i, 0)),
            pl.BlockSpec((_D, _EP), lambda i: (0, 0)),
            pl.BlockSpec((1, _EP), lambda i: (0, 0)),
        ],
        out_specs=[
            pl.BlockSpec((1, 1, _RB), lambda i: (i, 0, 0)),
            pl.BlockSpec((1, 1, _RB), lambda i: (i, 0, 0)),
            pl.BlockSpec((1, 1, _RB), lambda i: (i, 0, 0)),
            pl.BlockSpec((1, _EP), lambda i: (0, 0)),
            pl.BlockSpec((1, _EP), lambda i: (0, 0)),
        ],
        out_shape=[
            jax.ShapeDtypeStruct((_NRB, 1, _RB), jnp.int32),
            jax.ShapeDtypeStruct((_NRB, 1, _RB), jnp.float32),
            jax.ShapeDtypeStruct((_NRB, 1, _RB), jnp.int32),
            jax.ShapeDtypeStruct((1, _EP), jnp.float32),
            jax.ShapeDtypeStruct((1, _EP), jnp.float32),
        ],
        scratch_shapes=[pltpu.VMEM((8, _EP), jnp.float32)],
    )(x_flat, wp, bp)
    return (eid3.reshape(_N), gate3.reshape(_N), rank3.reshape(_N), cnt,
            aux[0, 0])


def _build_work_table(counts):
    off = jnp.concatenate(
        [jnp.zeros((1,), jnp.int32), jnp.cumsum(counts, dtype=jnp.int32)])
    bidx = jnp.arange(_NBLK, dtype=jnp.int32)[:, None]
    eidx = jnp.arange(_E, dtype=jnp.int32)[None, :]
    blk_lo = bidx * _TB
    blk_hi = blk_lo + _TB
    g_lo = off[:-1][None, :]
    g_hi = off[1:][None, :]
    active = (g_lo < blk_hi) & (g_hi > blk_lo)
    firsts = active & (jnp.cumsum(active.astype(jnp.int32), axis=1) == 1)
    afl = active.reshape(-1)
    rank = jnp.cumsum(afl.astype(jnp.int32)) - 1
    slot = jnp.where(afl, rank, _NWORK)
    bb = jnp.broadcast_to(bidx, (_NBLK, _E)).reshape(-1)
    ee = jnp.broadcast_to(eidx, (_NBLK, _E)).reshape(-1)
    last_e = (jnp.searchsorted(off, _N - 1, side='right') - 1).astype(jnp.int32)
    wb = jnp.full((_NWORK,), _NBLK - 1, jnp.int32).at[slot].set(bb, mode='drop')
    wef = jnp.full((_NWORK,), last_e, jnp.int32).at[slot].set(ee, mode='drop')
    wem = jnp.full((_NWORK,), -1, jnp.int32).at[slot].set(ee, mode='drop')
    wfirst = jnp.zeros((_NWORK,), jnp.int32).at[slot].set(
        firsts.reshape(-1).astype(jnp.int32), mode='drop')
    return jnp.stack([wb, wef, wem, wfirst])


def _ffn_body(tbl_ref, xs_ref, w1_ref, b1_ref, w2_ref, b2_ref, eid_ref,
              gate_ref, out_ref):
    w = pl.program_id(0)
    emask = tbl_ref[2, w]
    first = tbl_ref[3, w]

    @pl.when(emask >= 0)
    def _():
        xb = xs_ref[...]
        h = jnp.dot(xb, w1_ref[0], preferred_element_type=jnp.float32)
        h = jnp.maximum(h + b1_ref[0, 0], 0.0)
        y = jnp.dot(h, w2_ref[0], preferred_element_type=jnp.float32)
        y = y + b2_ref[0, 0]
        wt = jnp.where(eid_ref[0, 0, :] == emask, gate_ref[0, 0, :], 0.0)
        contrib = y * wt[:, None]

        @pl.when(first == 1)
        def _():
            out_ref[...] = contrib

        @pl.when(first == 0)
        def _():
            out_ref[...] += contrib


def _run_ffn(xs, W1, b1, W2, b2, eid_s, gate_s, tbl):
    grid_spec = pltpu.PrefetchScalarGridSpec(
        num_scalar_prefetch=1,
        grid=(_NWORK,),
        in_specs=[
            pl.BlockSpec((_TB, _D), lambda w, tbl: (tbl[0, w], 0)),
            pl.BlockSpec((1, _D, _H), lambda w, tbl: (tbl[1, w], 0, 0)),
            pl.BlockSpec((1, 1, _H), lambda w, tbl: (tbl[1, w], 0, 0)),
            pl.BlockSpec((1, _H, _D), lambda w, tbl: (tbl[1, w], 0, 0)),
            pl.BlockSpec((1, 1, _D), lambda w, tbl: (tbl[1, w], 0, 0)),
            pl.BlockSpec((1, 1, _TB), lambda w, tbl: (tbl[0, w], 0, 0)),
            pl.BlockSpec((1, 1, _TB), lambda w, tbl: (tbl[0, w], 0, 0)),
        ],
        out_specs=pl.BlockSpec((_TB, _D), lambda w, tbl: (tbl[0, w], 0)),
    )
    return pl.pallas_call(
        _ffn_body,
        grid_spec=grid_spec,
        out_shape=jax.ShapeDtypeStruct((_N, _D), jnp.float32),
    )(tbl, xs, W1, b1.reshape(_E, 1, _H), W2, b2.reshape(_E, 1, _D),
      eid_s.reshape(_NBLK, 1, _TB), gate_s.reshape(_NBLK, 1, _TB))


def _sc_gather_body(table_hbm, idx_hbm, out_hbm, idx_v, rows_v, sem):
    wid = lax.axis_index("s") * _NC + lax.axis_index("c")
    base = wid * _RPW
    for j in range(_NCHK):
        b = base + j * _GCH
        pltpu.sync_copy(idx_hbm.at[pl.ds(b, _GCH)], idx_v)
        pltpu.async_copy(table_hbm.at[idx_v], rows_v, sem).wait()
        pltpu.sync_copy(rows_v, out_hbm.at[pl.ds(b, _GCH)])


def _sc_gather_rows(table, idx):
    k = functools.partial(
        pl.kernel,
        mesh=plsc.VectorSubcoreMesh(core_axis_name="c", subcore_axis_name="s",
                                    num_cores=_NC, num_subcores=_NS),
        out_type=jax.ShapeDtypeStruct((_N, _D), jnp.float32),
        scratch_types=[
            pltpu.VMEM((_GCH,), jnp.int32),
            pltpu.VMEM((_GCH, _D), jnp.float32),
            pltpu.SemaphoreType.DMA,
        ],
    )(_sc_gather_body)
    return k(table, idx)


def _cls_body(mo_ref, lw_ref, lb_ref, out_ref, acc_ref):
    i = pl.program_id(0)

    @pl.when(i == 0)
    def _():
        acc_ref[...] = jnp.zeros_like(acc_ref)

    acc_ref[...] += jnp.sum(mo_ref[...], axis=1)

    @pl.when(i == _NSB - 1)
    def _():
        pooled = acc_ref[...] * (1.0 / _S)
        out_ref[...] = (jnp.dot(pooled, lw_ref[...],
                                preferred_element_type=jnp.float32)
                        + lb_ref[...])


def _run_classifier(moe_out, lin_w, lin_b):
    return pl.pallas_call(
        _cls_body,
        grid=(_NSB,),
        in_specs=[
            pl.BlockSpec((_B, _SB, _D), lambda i: (0, i, 0)),
            pl.BlockSpec((_D, _C), lambda i: (0, 0)),
            pl.BlockSpec((1, _C), lambda i: (0, 0)),
        ],
        out_specs=pl.BlockSpec((_B, _C), lambda i: (0, 0)),
        out_shape=jax.ShapeDtypeStruct((_B, _C), jnp.float32),
        scratch_shapes=[pltpu.VMEM((_B, _D), jnp.float32)],
    )(moe_out, lin_w, lin_b.reshape(1, _C))


def kernel(x, router_w, router_b, W1, b1, W2, b2, lin_w, lin_b):
    x_flat = x.reshape(_N, _D)
    eid, gate, rank, cnt, aux = _run_router(x_flat, router_w, router_b)

    counts = cnt[0, :_E].astype(jnp.int32)
    off = jnp.concatenate(
        [jnp.zeros((1,), jnp.int32), jnp.cumsum(counts, dtype=jnp.int32)])
    inv_perm = jnp.take(off[:_E], eid) + rank
    perm = jnp.zeros((_N,), jnp.int32).at[inv_perm].set(
        jnp.arange(_N, dtype=jnp.int32))
    tbl = _build_work_table(counts)

    xs = _sc_gather_rows(x_flat, perm)
    eid_s = jnp.take(eid, perm)
    gate_s = jnp.take(gate, perm)

    ys = _run_ffn(xs, W1, b1, W2, b2, eid_s, gate_s, tbl)

    moe_flat = _sc_gather_rows(ys, inv_perm)
    moe_out = moe_flat.reshape(_B, _S, _D)

    logits = _run_classifier(moe_out, lin_w, lin_b)
    return (logits, moe_out, aux)


# speedup vs baseline: 3.0102x; 1.0577x over previous
import functools

import jax
import jax.numpy as jnp
from jax import lax
from jax.experimental import pallas as pl
from jax.experimental.pallas import tpu as pltpu
from jax.experimental.pallas import tpu_sc as plsc

_B, _S, _D, _H, _E, _C = 2, 2048, 1024, 2048, 4, 1000
_N = _B * _S
_EP = 128
_RB = 512
_NRB = _N // _RB
_TB = 256
_NBLK = _N // _TB
_NWORK = _NBLK + _E - 1
_SB = 256
_NSB = _S // _SB

_NC = 2
_NS = 16
_NW = _NC * _NS
_RPW = _N // _NW
_GCH = 32
_NCHK = _RPW // _GCH


def _router_body(x_ref, w_ref, b_ref, eid_ref, gate_ref, rank_ref, cnt_ref,
                 aux_ref, acc_ref):
    i = pl.program_id(0)

    @pl.when(i == 0)
    def _():
        acc_ref[...] = jnp.zeros_like(acc_ref)

    xb = x_ref[...]
    logits = jnp.dot(xb, w_ref[...], preferred_element_type=jnp.float32)
    logits = logits + b_ref[...]
    m = jnp.max(logits, axis=-1, keepdims=True)
    p = jnp.exp(logits - m)
    s = jnp.sum(p, axis=-1, keepdims=True)
    probs = p / s
    eid = jnp.argmax(logits, axis=-1).astype(jnp.int32)
    gate = 1.0 / s[:, 0]
    eid_ref[0, 0, :] = eid
    gate_ref[0, 0, :] = gate
    onehot = (lax.broadcasted_iota(jnp.int32, (_RB, _EP), 1)
              == eid[:, None]).astype(jnp.float32)
    tri = (lax.broadcasted_iota(jnp.int32, (_RB, _RB), 0)
           > lax.broadcasted_iota(jnp.int32, (_RB, _RB), 1)).astype(jnp.float32)
    prefix = jnp.dot(tri, onehot, preferred_element_type=jnp.float32)
    rank = jnp.sum(onehot * (acc_ref[1:2, :] + prefix), axis=1)
    rank_ref[0, 0, :] = rank.astype(jnp.int32)
    acc_ref[0:1, :] += jnp.sum(probs, axis=0, keepdims=True)
    acc_ref[1:2, :] += jnp.sum(onehot, axis=0, keepdims=True)

    @pl.when(i == _NRB - 1)
    def _():
        cnt_ref[...] = acc_ref[1:2, :]
        aux = (_E / (_N * _N)) * jnp.sum(acc_ref[0:1, :] * acc_ref[1:2, :])
        aux_ref[...] = aux * jnp.ones((1, _EP), jnp.float32)


def _run_router(x_flat, router_w, router_b):
    wp = jnp.pad(router_w, ((0, 0), (0, _EP - _E)))
    bp = jnp.full((1, _EP), -jnp.inf, jnp.float32).at[0, :_E].set(router_b)
    eid3, gate3, rank3, cnt, aux = pl.pallas_call(
        _router_body,
        grid=(_NRB,),
        in_specs=[
            pl.BlockSpec((_RB, _D), lambda i: (i, 0)),
            pl.BlockSpec((_D, _EP), lambda i: (0, 0)),
            pl.BlockSpec((1, _EP), lambda i: (0, 0)),
        ],
        out_specs=[
            pl.BlockSpec((1, 1, _RB), lambda i: (i, 0, 0)),
            pl.BlockSpec((1, 1, _RB), lambda i: (i, 0, 0)),
            pl.BlockSpec((1, 1, _RB), lambda i: (i, 0, 0)),
            pl.BlockSpec((1, _EP), lambda i: (0, 0)),
            pl.BlockSpec((1, _EP), lambda i: (0, 0)),
        ],
        out_shape=[
            jax.ShapeDtypeStruct((_NRB, 1, _RB), jnp.int32),
            jax.ShapeDtypeStruct((_NRB, 1, _RB), jnp.float32),
            jax.ShapeDtypeStruct((_NRB, 1, _RB), jnp.int32),
            jax.ShapeDtypeStruct((1, _EP), jnp.float32),
            jax.ShapeDtypeStruct((1, _EP), jnp.float32),
        ],
        scratch_shapes=[pltpu.VMEM((8, _EP), jnp.float32)],
    )(x_flat, wp, bp)
    return (eid3.reshape(_N), gate3.reshape(_N), rank3.reshape(_N), cnt,
            aux[0, 0])


def _build_work_table(off):
    bidx = jnp.arange(_NBLK, dtype=jnp.int32)[:, None]
    eidx = jnp.arange(_E, dtype=jnp.int32)[None, :]
    blk_lo = bidx * _TB
    blk_hi = blk_lo + _TB
    g_lo = off[:-1][None, :]
    g_hi = off[1:][None, :]
    active = (g_lo < blk_hi) & (g_hi > blk_lo)
    firsts = active & (jnp.cumsum(active.astype(jnp.int32), axis=1) == 1)
    lo = jnp.clip(g_lo - blk_lo, 0, _TB)
    hi = jnp.clip(g_hi - blk_lo, 0, _TB)
    afl = active.reshape(-1)
    rank = jnp.cumsum(afl.astype(jnp.int32)) - 1
    slot = jnp.where(afl, rank, _NWORK)
    bb = jnp.broadcast_to(bidx, (_NBLK, _E))
    ee = jnp.broadcast_to(eidx, (_NBLK, _E))
    last_e = jnp.sum((off[1:] < _N).astype(jnp.int32))
    defaults = jnp.stack([
        jnp.full((_NWORK,), _NBLK - 1, jnp.int32),
        jnp.full((_NWORK,), last_e, jnp.int32),
        jnp.full((_NWORK,), -1, jnp.int32),
        jnp.zeros((_NWORK,), jnp.int32),
        jnp.zeros((_NWORK,), jnp.int32),
        jnp.zeros((_NWORK,), jnp.int32),
    ])
    vals = jnp.stack([bb.reshape(-1), ee.reshape(-1), ee.reshape(-1),
                      firsts.reshape(-1).astype(jnp.int32),
                      lo.reshape(-1), hi.reshape(-1)])
    return defaults.at[:, slot].set(vals, mode='drop')


def _ffn_body(tbl_ref, xs_ref, w1_ref, b1_ref, w2_ref, b2_ref,
              gate_ref, bw_ref, out_ref, psum_ref, pacc_ref):
    w = pl.program_id(0)
    emask = tbl_ref[2, w]
    first = tbl_ref[3, w]

    @pl.when(w == 0)
    def _():
        pacc_ref[...] = jnp.zeros_like(pacc_ref)

    @pl.when(emask >= 0)
    def _():
        xb = xs_ref[...]
        h = jnp.dot(xb, w1_ref[0], preferred_element_type=jnp.float32)
        h = jnp.maximum(h + b1_ref[0, 0], 0.0)
        y = jnp.dot(h, w2_ref[0], preferred_element_type=jnp.float32)
        y = y + b2_ref[0, 0]
        ri = lax.broadcasted_iota(jnp.int32, (1, _TB), 1)[0]
        sel = (ri >= tbl_ref[4, w]) & (ri < tbl_ref[5, w])
        wt = jnp.where(sel, gate_ref[0, 0, :], 0.0)
        contrib = y * wt[:, None]
        s_all = jnp.sum(contrib, axis=0, keepdims=True)
        s_b0 = jnp.sum(contrib * bw_ref[0, 0, :][:, None], axis=0,
                       keepdims=True)
        pacc_ref[0:1, :] += s_b0
        pacc_ref[1:2, :] += s_all - s_b0

        @pl.when(first == 1)
        def _():
            out_ref[...] = contrib

        @pl.when(first == 0)
        def _():
            out_ref[...] += contrib

    @pl.when(w == _NWORK - 1)
    def _():
        psum_ref[...] = pacc_ref[...]


def _run_ffn(xs, W1, b1, W2, b2, gate_s, bw_s, tbl):
    grid_spec = pltpu.PrefetchScalarGridSpec(
        num_scalar_prefetch=1,
        grid=(_NWORK,),
        in_specs=[
            pl.BlockSpec((_TB, _D), lambda w, tbl: (tbl[0, w], 0)),
            pl.BlockSpec((1, _D, _H), lambda w, tbl: (tbl[1, w], 0, 0)),
            pl.BlockSpec((1, 1, _H), lambda w, tbl: (tbl[1, w], 0, 0)),
            pl.BlockSpec((1, _H, _D), lambda w, tbl: (tbl[1, w], 0, 0)),
            pl.BlockSpec((1, 1, _D), lambda w, tbl: (tbl[1, w], 0, 0)),
            pl.BlockSpec((1, 1, _TB), lambda w, tbl: (tbl[0, w], 0, 0)),
            pl.BlockSpec((1, 1, _TB), lambda w, tbl: (tbl[0, w], 0, 0)),
        ],
        out_specs=[
            pl.BlockSpec((_TB, _D), lambda w, tbl: (tbl[0, w], 0)),
            pl.BlockSpec((8, _D), lambda w, tbl: (0, 0)),
        ],
        scratch_shapes=[pltpu.VMEM((8, _D), jnp.float32)],
    )
    return pl.pallas_call(
        _ffn_body,
        grid_spec=grid_spec,
        out_shape=[
            jax.ShapeDtypeStruct((_N, _D), jnp.float32),
            jax.ShapeDtypeStruct((8, _D), jnp.float32),
        ],
    )(tbl, xs, W1, b1.reshape(_E, 1, _H), W2, b2.reshape(_E, 1, _D),
      gate_s.reshape(_NBLK, 1, _TB), bw_s.reshape(_NBLK, 1, _TB))


def _sc_gather_body(table_hbm, idx_hbm, out_hbm, idx_v, rows_v, sem0, sem1):
    wid = lax.axis_index("s") * _NC + lax.axis_index("c")
    base = wid * _RPW
    pltpu.sync_copy(idx_hbm.at[pl.ds(base, _RPW)], idx_v)
    sems = (sem0, sem1)
    handles = [None, None]
    for j in range(_NCHK):
        handles[j % 2] = pltpu.async_copy(
            table_hbm.at[idx_v.at[pl.ds(j * _GCH, _GCH)]], rows_v.at[j % 2],
            sems[j % 2])
        if j > 0:
            handles[(j - 1) % 2].wait()
            pltpu.sync_copy(rows_v.at[(j - 1) % 2],
                            out_hbm.at[pl.ds(base + (j - 1) * _GCH, _GCH)])
    handles[(_NCHK - 1) % 2].wait()
    pltpu.sync_copy(rows_v.at[(_NCHK - 1) % 2],
                    out_hbm.at[pl.ds(base + (_NCHK - 1) * _GCH, _GCH)])


def _sc_gather_rows(table, idx):
    k = functools.partial(
        pl.kernel,
        mesh=plsc.VectorSubcoreMesh(core_axis_name="c", subcore_axis_name="s",
                                    num_cores=_NC, num_subcores=_NS),
        out_type=jax.ShapeDtypeStruct((_N, _D), jnp.float32),
        scratch_types=[
            pltpu.VMEM((_RPW,), jnp.int32),
            pltpu.VMEM((2, _GCH, _D), jnp.float32),
            pltpu.SemaphoreType.DMA,
            pltpu.SemaphoreType.DMA,
        ],
    )(_sc_gather_body)
    return k(table, idx)


def _sc_scatter_body(src_hbm, idx_hbm, out_hbm, idx_v, rows_v, sem0, sem1):
    wid = lax.axis_index("s") * _NC + lax.axis_index("c")
    base = wid * _RPW
    pltpu.sync_copy(idx_hbm.at[wid], idx_v)
    sems = (sem0, sem1)
    handles = [None, None]
    for j in range(_NCHK):
        if j >= 2:
            handles[j % 2].wait()
        pltpu.sync_copy(src_hbm.at[pl.ds(base + j * _GCH, _GCH)],
                        rows_v.at[j % 2])
        handles[j % 2] = pltpu.async_copy(
            rows_v.at[j % 2], out_hbm.at[idx_v.at[j]], sems[j % 2])
    handles[0].wait()
    handles[1].wait()


def _sc_scatter_rows(src, idx):
    k = functools.partial(
        pl.kernel,
        mesh=plsc.VectorSubcoreMesh(core_axis_name="c", subcore_axis_name="s",
                                    num_cores=_NC, num_subcores=_NS),
        out_type=jax.ShapeDtypeStruct((_N, _D), jnp.float32),
        scratch_types=[
            pltpu.VMEM((_NCHK, _GCH), jnp.int32),
            pltpu.VMEM((2, _GCH, _D), jnp.float32),
            pltpu.SemaphoreType.DMA,
            pltpu.SemaphoreType.DMA,
        ],
    )(_sc_scatter_body)
    return k(src, idx.reshape(_NW, _NCHK, _GCH))


def _cls_body(ps_ref, lw_ref, lb_ref, out_ref):
    pooled = ps_ref[0:_B, :] * (1.0 / _S)
    out_ref[...] = (jnp.dot(pooled, lw_ref[...],
                            preferred_element_type=jnp.float32)
                    + lb_ref[...])


def _run_classifier(psum, lin_w, lin_b):
    return pl.pallas_call(
        _cls_body,
        out_shape=jax.ShapeDtypeStruct((_B, _C), jnp.float32),
    )(psum, lin_w, lin_b.reshape(1, _C))


def kernel(x, router_w, router_b, W1, b1, W2, b2, lin_w, lin_b):
    x_flat = x.reshape(_N, _D)
    eid, gate, rank, cnt, aux = _run_router(x_flat, router_w, router_b)

    counts = cnt[0, :_E].astype(jnp.int32)
    off = jnp.concatenate(
        [jnp.zeros((1,), jnp.int32), jnp.cumsum(counts, dtype=jnp.int32)])
    dst = jnp.take(off[:_E], eid) + rank
    tbl = _build_work_table(off)

    xs = _sc_scatter_rows(x_flat, dst)
    gate_s = jnp.zeros((_N,), jnp.float32).at[dst].set(gate)
    bw_s = jnp.zeros((_N,), jnp.float32).at[dst].set(
        (jnp.arange(_N, dtype=jnp.int32) < _S).astype(jnp.float32))

    ys, psum = _run_ffn(xs, W1, b1, W2, b2, gate_s, bw_s, tbl)

    moe_flat = _sc_gather_rows(ys, dst)
    moe_out = moe_flat.reshape(_B, _S, _D)

    logits = _run_classifier(psum, lin_w, lin_b)
    return (logits, moe_out, aux)
